# baseline (speedup 1.0000x reference)
import functools

import jax
import jax.numpy as jnp
import numpy as np
from jax import lax
from jax.experimental import pallas as pl
from jax.experimental.pallas import tpu as pltpu
from jax.experimental.pallas import tpu_sc as plsc

P = 128
H = 64
ALPHA = 0.5
BB = 64

_TINY = np.float32(np.finfo(np.float32).tiny)
_K0 = 0
_K1 = 1234
_K2 = _K0 ^ _K1 ^ 0x1BD11BDA


def _rotl(x, d):
    return (x << jnp.uint32(d)) | (x >> jnp.uint32(32 - d))


def _threefry2x32(x1):
    ks = (jnp.uint32(_K0), jnp.uint32(_K1), jnp.uint32(_K2))
    rots = ((13, 15, 26, 6), (17, 29, 16, 24))
    x0 = jnp.zeros_like(x1) + ks[0]
    x1 = x1 + ks[1]
    for r in range(5):
        for rot in rots[r % 2]:
            x0 = x0 + x1
            x1 = _rotl(x1, rot)
            x1 = x0 ^ x1
        x0 = x0 + ks[(r + 1) % 3]
        x1 = x1 + ks[(r + 2) % 3] + jnp.uint32(r + 1)
    return x0 ^ x1


def _sample_kernel(pbt_ref, tprob_ref, out_idx_ref, out_prob_ref):
    j0 = pl.program_id(0)

    pbt = pbt_ref[...]
    rp = ALPHA * jnp.exp(pbt) + (1.0 - ALPHA) / P
    logits = jnp.log(rp)

    shp = (BB, P, P)
    b_i = lax.broadcasted_iota(jnp.int32, shp, 0) + j0 * BB
    c_i = lax.broadcasted_iota(jnp.int32, shp, 1)
    j_i = lax.broadcasted_iota(jnp.int32, shp, 2)
    cnt = (b_i * (P * P) + j_i * P + c_i).astype(jnp.uint32)
    bits = _threefry2x32(cnt)
    fbits = (bits >> jnp.uint32(9)) | jnp.uint32(0x3F800000)
    floats = lax.bitcast_convert_type(fbits, jnp.float32) - 1.0
    u = jnp.maximum(_TINY, floats + _TINY)
    score = -jnp.log(-jnp.log(u)) + logits[:, :, None]

    mx = jnp.max(score, axis=1, keepdims=True)
    cand = jnp.where(score == mx, c_i, P)
    idx = jnp.min(cand, axis=1)
    out_idx_ref[...] = idx

    idx_t = idx.T
    c2 = lax.broadcasted_iota(jnp.int32, (P, BB, P), 2)
    oh = (idx_t[:, :, None] == c2).astype(jnp.float32).reshape(P * BB, P)
    t = tprob_ref[...]
    et = jnp.exp(t)
    ft = jnp.log(et / (ALPHA * et + (1.0 - ALPHA) / P))
    pvals = jax.lax.dot_general(
        oh, ft,
        dimension_numbers=(((1,), (1,)), ((), ())),
        precision=jax.lax.Precision.HIGHEST,
        preferred_element_type=jnp.float32)
    pvals = pvals.reshape(P, BB).T
    m = jnp.max(pvals, axis=-1, keepdims=True)
    lse = jnp.log(jnp.sum(jnp.exp(pvals - m), axis=-1, keepdims=True)) + m
    out_prob_ref[...] = pvals - lse


_SC_CORES = 2
_SC_SUBCORES = 16


def _make_sc_gather(n_rows):
    nw = _SC_CORES * _SC_SUBCORES
    rows_per_w = n_rows // nw
    chunk = 1024
    n_chunks = rows_per_w // chunk
    mesh = plsc.VectorSubcoreMesh(
        core_axis_name="c", subcore_axis_name="s",
        num_cores=_SC_CORES, num_subcores=_SC_SUBCORES)

    @functools.partial(
        pl.kernel, mesh=mesh,
        compiler_params=pltpu.CompilerParams(use_tc_tiling_on_sc=False),
        out_type=jax.ShapeDtypeStruct((n_rows, H), jnp.float32),
        scratch_types=[
            pltpu.VMEM((chunk,), jnp.int32),
            pltpu.VMEM((chunk, H), jnp.float32),
            pltpu.SemaphoreType.DMA,
        ],
    )
    def sc_gather(table_hbm, idx_hbm, out_hbm, idx_v, rows_v, sem):
        wid = lax.axis_index("s") * _SC_CORES + lax.axis_index("c")
        base = wid * rows_per_w
        for c in range(n_chunks):
            pltpu.sync_copy(idx_hbm.at[pl.ds(base + c * chunk, chunk)], idx_v)
            pltpu.async_copy(table_hbm.at[idx_v], rows_v, sem).wait()
            pltpu.sync_copy(rows_v, out_hbm.at[pl.ds(base + c * chunk, chunk)])

    return sc_gather


@jax.jit
def kernel(particles, prob):
    B = prob.shape[0] // P
    prob2d = prob.reshape(P, B)
    pbt = prob2d.T
    tprob = prob.reshape(-1)[:P].reshape(1, P)
    table = particles[:P]

    grid = (B // BB,)
    idx_bp, out_prob_t = pl.pallas_call(
        _sample_kernel,
        grid=grid,
        in_specs=[
            pl.BlockSpec((BB, P), lambda j: (j, 0)),
            pl.BlockSpec((1, P), lambda j: (0, 0)),
        ],
        out_specs=[
            pl.BlockSpec((BB, P), lambda j: (j, 0)),
            pl.BlockSpec((BB, P), lambda j: (j, 0)),
        ],
        out_shape=[
            jax.ShapeDtypeStruct((B, P), jnp.int32),
            jax.ShapeDtypeStruct((B, P), jnp.float32),
        ],
        compiler_params=pltpu.CompilerParams(
            dimension_semantics=("parallel",)),
    )(pbt, tprob)

    n_rows = P * B
    idx_pm = idx_bp.T.reshape(n_rows)
    particles_new = _make_sc_gather(n_rows)(table, idx_pm)
    return particles_new, out_prob_t.T


# speedup vs baseline: 1.4037x; 1.4037x over previous
import jax
import jax.numpy as jnp
import numpy as np
from jax import lax
from jax.experimental import pallas as pl
from jax.experimental.pallas import tpu as pltpu

P = 128
H = 64
ALPHA = 0.5
BB = 64

_TINY = np.float32(np.finfo(np.float32).tiny)
_K0 = 0
_K1 = 1234
_K2 = _K0 ^ _K1 ^ 0x1BD11BDA


def _rotl(x, d):
    return (x << jnp.uint32(d)) | (x >> jnp.uint32(32 - d))


def _threefry2x32(x1):
    ks = (jnp.uint32(_K0), jnp.uint32(_K1), jnp.uint32(_K2))
    rots = ((13, 15, 26, 6), (17, 29, 16, 24))
    x0 = jnp.zeros_like(x1) + ks[0]
    x1 = x1 + ks[1]
    for r in range(5):
        for rot in rots[r % 2]:
            x0 = x0 + x1
            x1 = _rotl(x1, rot)
            x1 = x0 ^ x1
        x0 = x0 + ks[(r + 1) % 3]
        x1 = x1 + ks[(r + 2) % 3] + jnp.uint32(r + 1)
    return x0 ^ x1


def _resample_kernel(pbt_ref, tprob_ref, table_ref, out_part_ref, out_prob_ref):
    j0 = pl.program_id(0)

    pbt = pbt_ref[...]
    rp = ALPHA * jnp.exp(pbt) + (1.0 - ALPHA) / P
    logits = jnp.log(rp)

    shp = (BB, P, P)
    b_i = lax.broadcasted_iota(jnp.int32, shp, 0) + j0 * BB
    c_i = lax.broadcasted_iota(jnp.int32, shp, 1)
    j_i = lax.broadcasted_iota(jnp.int32, shp, 2)
    cnt = (b_i * (P * P) + j_i * P + c_i).astype(jnp.uint32)
    bits = _threefry2x32(cnt)
    fbits = (bits >> jnp.uint32(9)) | jnp.uint32(0x3F800000)
    floats = lax.bitcast_convert_type(fbits, jnp.float32) - 1.0
    u = jnp.maximum(_TINY, floats + _TINY)
    score = -jnp.log(-jnp.log(u)) + logits[:, :, None]

    mx = jnp.max(score, axis=1, keepdims=True)
    cand = jnp.where(score == mx, c_i, P)
    idx = jnp.min(cand, axis=1)

    idx_t = idx.T
    c2 = lax.broadcasted_iota(jnp.int32, (P, BB, P), 2)
    oh = (idx_t[:, :, None] == c2).astype(jnp.float32).reshape(P * BB, P)

    gathered = jax.lax.dot_general(
        oh, table_ref[...],
        dimension_numbers=(((1,), (0,)), ((), ())),
        precision=jax.lax.Precision.HIGHEST,
        preferred_element_type=jnp.float32)
    out_part_ref[...] = gathered.reshape(P, BB, H)

    t = tprob_ref[...]
    et = jnp.exp(t)
    ft = jnp.log(et / (ALPHA * et + (1.0 - ALPHA) / P))
    pvals = jax.lax.dot_general(
        oh, ft,
        dimension_numbers=(((1,), (1,)), ((), ())),
        precision=jax.lax.Precision.HIGHEST,
        preferred_element_type=jnp.float32)
    pvals = pvals.reshape(P, BB).T
    m = jnp.max(pvals, axis=-1, keepdims=True)
    lse = jnp.log(jnp.sum(jnp.exp(pvals - m), axis=-1, keepdims=True)) + m
    out_prob_ref[...] = pvals - lse


@jax.jit
def kernel(particles, prob):
    B = prob.shape[0] // P
    prob2d = prob.reshape(P, B)
    pbt = prob2d.T
    tprob = prob.reshape(-1)[:P].reshape(1, P)
    table = particles[:P]

    grid = (B // BB,)
    out_part, out_prob_t = pl.pallas_call(
        _resample_kernel,
        grid=grid,
        in_specs=[
            pl.BlockSpec((BB, P), lambda j: (j, 0)),
            pl.BlockSpec((1, P), lambda j: (0, 0)),
            pl.BlockSpec((P, H), lambda j: (0, 0)),
        ],
        out_specs=[
            pl.BlockSpec((P, BB, H), lambda j: (0, j, 0)),
            pl.BlockSpec((BB, P), lambda j: (j, 0)),
        ],
        out_shape=[
            jax.ShapeDtypeStruct((P, B, H), jnp.float32),
            jax.ShapeDtypeStruct((B, P), jnp.float32),
        ],
        compiler_params=pltpu.CompilerParams(
            dimension_semantics=("parallel",)),
    )(pbt, tprob, table)

    return out_part.reshape(P * B, H), out_prob_t.T
